# baseline (speedup 1.0000x reference)
import functools

import jax
import jax.numpy as jnp
from jax import lax
from jax.experimental import pallas as pl
from jax.experimental.pallas import tpu as pltpu
from jax.experimental.pallas import tpu_sc as plsc

NC = 2
NS = 16
NW = NC * NS
CH = 128
KJ = 8
BN = 1024


def _sc_mesh():
    return plsc.VectorSubcoreMesh(core_axis_name="c", subcore_axis_name="s",
                                  num_cores=NC, num_subcores=NS)


def _make_deg_kernel(Ep, Np):
    CPT = Ep // CH // NW
    SUP = CPT // KJ
    SUPP = SUP // 2
    SP = Np // NS

    @functools.partial(
        pl.kernel,
        out_type=[jax.ShapeDtypeStruct((Np,), jnp.float32)] * NC,
        mesh=_sc_mesh(),
        scratch_types=[
            pltpu.VMEM((KJ, CH), jnp.int32),
            pltpu.VMEM((KJ, CH), jnp.int32),
            pltpu.VMEM((CH,), jnp.float32),
            pltpu.VMEM((SP,), jnp.float32),
            pltpu.VMEM_SHARED((Np,), jnp.float32),
            pltpu.SemaphoreType.DMA,
            pltpu.SemaphoreType.DMA,
        ],
        compiler_params=pltpu.CompilerParams(use_tc_tiling_on_sc=False),
        name="gcn_deg",
    )
    def degk(dst_hbm, out0_hbm, out1_hbm, idxe, idxd, ones_v, stage, acc,
             seme, semd):
        c = lax.axis_index("c")
        s = lax.axis_index("s")
        wid = c * NS + s
        base = wid * CPT
        for i in range(CH // 16):
            ones_v[pl.ds(i * 16, 16)] = jnp.ones((16,), jnp.float32)

        def zbody(i, carry):
            stage[pl.ds(i * 16, 16)] = jnp.zeros((16,), jnp.float32)
            return carry

        lax.fori_loop(0, SP // 16, zbody, 0)
        pltpu.sync_copy(stage, acc.at[pl.ds(s * SP, SP)])
        plsc.subcore_barrier()

        pltpu.sync_copy(dst_hbm.at[pl.ds(base, KJ)], idxe)

        def drain(sem):
            pltpu.make_async_copy(out0_hbm.at[pl.ds(0, KJ * CH)],
                                  stage.at[pl.ds(0, KJ * CH)], sem).wait()

        def body(p, carry):
            for j in range(KJ):
                pltpu.async_copy(ones_v, acc.at[idxe.at[j]], seme, add=True)
            pltpu.sync_copy(dst_hbm.at[pl.ds(base + (2 * p + 1) * KJ, KJ)],
                            idxd)
            drain(seme)
            for j in range(KJ):
                pltpu.async_copy(ones_v, acc.at[idxd.at[j]], semd, add=True)

            @pl.when(p < SUPP - 1)
            def _():
                pltpu.sync_copy(
                    dst_hbm.at[pl.ds(base + (2 * p + 2) * KJ, KJ)], idxe)

            drain(semd)
            return carry

        lax.fori_loop(0, SUPP, body, 0)
        plsc.subcore_barrier()
        pltpu.sync_copy(acc.at[pl.ds(s * SP, SP)], stage)

        @pl.when(c == 0)
        def _():
            pltpu.sync_copy(stage, out0_hbm.at[pl.ds(s * SP, SP)])

        @pl.when(c == 1)
        def _():
            pltpu.sync_copy(stage, out1_hbm.at[pl.ds(s * SP, SP)])

    return degk


def _make_scatter_kernel(D, Ep, Np, name):
    KJ4 = 4
    CPT = Ep // CH // NW
    NQ = CPT // (4 * KJ4)
    SP = Np // NS
    SPH = SP // 16
    RB = KJ4 * CH

    @functools.partial(
        pl.kernel,
        out_type=[jax.ShapeDtypeStruct((Np, D), jnp.float32)] * NC,
        mesh=_sc_mesh(),
        scratch_types=[
            pltpu.VMEM((2 * KJ4, CH), jnp.int32),
            pltpu.VMEM((2 * KJ4, CH), jnp.int32),
            pltpu.VMEM((2 * KJ4, CH), jnp.int32),
            pltpu.VMEM((2 * KJ4, CH), jnp.int32),
            pltpu.VMEM((RB, D), jnp.float32),
            pltpu.VMEM((RB, D), jnp.float32),
            pltpu.VMEM((SPH, D), jnp.float32),
            pltpu.VMEM_SHARED((Np, D), jnp.float32),
            pltpu.SemaphoreType.DMA,
            pltpu.SemaphoreType.DMA,
            pltpu.SemaphoreType.DMA,
            pltpu.SemaphoreType.DMA,
        ],
        compiler_params=pltpu.CompilerParams(use_tc_tiling_on_sc=False),
        name=name,
    )
    def scat(src_hbm, dst_hbm, tbl_hbm, zeros_hbm, out0_hbm, out1_hbm,
             sia, dia, sib, dib, r0, r1, stage, acc,
             gsa, gsb, ssa, ssb):
        c = lax.axis_index("c")
        s = lax.axis_index("s")
        wid = c * NS + s
        base = wid * CPT

        def ibody(p, carry):
            sl = pl.ds(s * SP + p * SPH, SPH)
            pltpu.sync_copy(zeros_hbm.at[sl], stage)
            pltpu.sync_copy(stage, acc.at[sl])
            return carry

        lax.fori_loop(0, 16, ibody, 0)
        plsc.subcore_barrier()

        def gather_fire(si, half, rows, sem):
            for j in range(KJ4):
                pltpu.async_copy(tbl_hbm.at[si.at[half * KJ4 + j]],
                                 rows.at[pl.ds(j * CH, CH)], sem)

        def scatter_fire(di, half, rows, sem):
            for j in range(KJ4):
                pltpu.async_copy(rows.at[pl.ds(j * CH, CH)],
                                 acc.at[di.at[half * KJ4 + j]], sem, add=True)

        def drain(sem, rows):
            pltpu.make_async_copy(tbl_hbm.at[pl.ds(0, RB)], rows, sem).wait()

        def load_pair(si, di, row0):
            pltpu.sync_copy(src_hbm.at[pl.ds(row0, 2 * KJ4)], si)
            pltpu.sync_copy(dst_hbm.at[pl.ds(row0, 2 * KJ4)], di)

        load_pair(sia, dia, base)
        gather_fire(sia, 0, r0, gsa)

        def body(q, carry):
            r = base + 4 * KJ4 * q
            drain(gsa, r0)
            scatter_fire(dia, 0, r0, ssa)

            @pl.when(q > 0)
            def _():
                drain(ssb, r1)
            gather_fire(sia, 1, r1, gsb)
            load_pair(sib, dib, r + 2 * KJ4)
            drain(gsb, r1)
            scatter_fire(dia, 1, r1, ssb)
            drain(ssa, r0)
            gather_fire(sib, 0, r0, gsa)
            drain(gsa, r0)
            scatter_fire(dib, 0, r0, ssa)
            drain(ssb, r1)
            gather_fire(sib, 1, r1, gsb)
            @pl.when(q < NQ - 1)
            def _():
                load_pair(sia, dia, r + 4 * KJ4)
            drain(gsb, r1)
            scatter_fire(dib, 1, r1, ssb)
            drain(ssa, r0)

            @pl.when(q < NQ - 1)
            def _():
                gather_fire(sia, 0, r0, gsa)
            return carry

        lax.fori_loop(0, NQ, body, 0)
        drain(ssb, r1)
        plsc.subcore_barrier()

        def obody(p, carry):
            sl = pl.ds(s * SP + p * SPH, SPH)
            pltpu.sync_copy(acc.at[sl], stage)

            @pl.when(c == 0)
            def _():
                pltpu.sync_copy(stage, out0_hbm.at[sl])

            @pl.when(c == 1)
            def _():
                pltpu.sync_copy(stage, out1_hbm.at[sl])
            return carry

        lax.fori_loop(0, 16, obody, 0)

    return scat


def _mm_body(xt_ref, w_ref, d0_ref, d1_ref, hs_ref):
    h = lax.dot_general(xt_ref[...], w_ref[...], (((0,), (0,)), ((), ())),
                        preferred_element_type=jnp.float32)
    deg = d0_ref[...] + d1_ref[...] + 1.0
    hs_ref[...] = h * lax.rsqrt(deg).reshape(-1, 1)


def _mid_body(a0_ref, a1_ref, hs_ref, d0_ref, d1_ref, b1t_ref, bd_ref,
              hs2_ref, *, npack):
    dinv16 = lax.rsqrt(d0_ref[...] + d1_ref[...] + 1.0)
    agg = a0_ref[...] + a1_ref[...] + hs_ref[...]
    z = jnp.maximum(agg * dinv16 + b1t_ref[...][None, :], 0.0)
    h2 = jnp.dot(z, bd_ref[...], preferred_element_type=jnp.float32)
    H16 = dinv16.shape[1] // npack
    O8 = h2.shape[1] // npack
    dinv8 = jnp.concatenate(
        [dinv16[:, a * H16:a * H16 + O8] for a in range(npack)], axis=1)
    hs2_ref[...] = h2 * dinv8


def _fin_body(a0_ref, a1_ref, hs2_ref, d0_ref, d1_ref, b2t_ref, gsum_ref,
              out_ref, *, nvalid, gw):
    dinv8 = lax.rsqrt(d0_ref[...] + d1_ref[...] + 1.0)
    agg = a0_ref[...] + a1_ref[...] + hs2_ref[...]
    logits = agg * dinv8 + b2t_ref[...][None, :]
    col = lax.broadcasted_iota(jnp.int32, logits.shape, 1)
    valid = (col % gw) < nvalid
    m = jnp.max(jnp.where(valid, logits, jnp.float32(-1e30)), axis=1,
                keepdims=True)
    e = jnp.where(valid, jnp.exp(logits - m), 0.0)
    gs = jnp.dot(e, gsum_ref[...], preferred_element_type=jnp.float32)
    out_ref[...] = logits - (jnp.log(gs) + m)


def kernel(x, edge_index, W1, b1, W2, b2):
    N, F = x.shape
    H = W1.shape[1]
    O = W2.shape[1]
    E = edge_index.shape[1]
    O8 = 8

    grain = NW * CH * KJ * 2
    Ep = ((E + grain - 1) // grain) * grain
    pad = Ep - E
    Np = ((N + NW + 1023) // 1024) * 1024
    nb = Np // BN

    src = edge_index[0]
    dst = edge_index[1]
    src_p = jnp.concatenate(
        [src, jnp.zeros((pad,), jnp.int32)]).reshape(Ep // CH, CH)
    dst_p = jnp.concatenate(
        [dst, N + (jnp.arange(pad, dtype=jnp.int32) % NW)]).reshape(Ep // CH, CH)
    zerosH = jnp.zeros((Np, H), jnp.float32)
    zerosO = jnp.zeros((Np, O8), jnp.float32)

    deg0, deg1 = _make_deg_kernel(Ep, Np)(dst_p)

    xt = jnp.swapaxes(x, 0, 1)
    hs = pl.pallas_call(
        _mm_body,
        grid=(nb,),
        in_specs=[
            pl.BlockSpec((F, BN), lambda i: (0, i)),
            pl.BlockSpec((F, H), lambda i: (0, 0)),
            pl.BlockSpec((BN,), lambda i: (i,)),
            pl.BlockSpec((BN,), lambda i: (i,)),
        ],
        out_specs=pl.BlockSpec((BN, H), lambda i: (i, 0)),
        out_shape=jax.ShapeDtypeStruct((Np, H), jnp.float32),
    )(xt, W1, deg0, deg1)

    a10, a11 = _make_scatter_kernel(H, Ep, Np, "gcn_scatter1")(
        src_p, dst_p, hs, zerosH)

    NP16 = Np * H // 128
    NP8 = Np * O8 // 128
    B16 = BN * H // 128
    B8 = BN * O8 // 128
    hsp = hs.reshape(Np * H).reshape(NP16, 128)
    a10p = a10.reshape(Np * H).reshape(NP16, 128)
    a11p = a11.reshape(Np * H).reshape(NP16, 128)
    d16_0 = jnp.repeat(deg0, H).reshape(NP16, 128)
    d16_1 = jnp.repeat(deg1, H).reshape(NP16, 128)

    W2p = jnp.pad(W2, ((0, 0), (0, O8 - O)))
    BD = jnp.kron(jnp.eye(128 // H, dtype=jnp.float32), W2p)
    b1t = jnp.tile(b1, 128 // H)
    BM = 896
    nbm = NP16 // BM
    hs2p = pl.pallas_call(
        functools.partial(_mid_body, npack=128 // H),
        grid=(nbm,),
        in_specs=[
            pl.BlockSpec((BM, 128), lambda i: (i, 0)),
            pl.BlockSpec((BM, 128), lambda i: (i, 0)),
            pl.BlockSpec((BM, 128), lambda i: (i, 0)),
            pl.BlockSpec((BM, 128), lambda i: (i, 0)),
            pl.BlockSpec((BM, 128), lambda i: (i, 0)),
            pl.BlockSpec((128,), lambda i: (0,)),
            pl.BlockSpec((128, 128 // H * O8), lambda i: (0, 0)),
        ],
        out_specs=pl.BlockSpec((BM, 128 // H * O8), lambda i: (i, 0)),
        out_shape=jax.ShapeDtypeStruct((NP16, 128 // H * O8), jnp.float32),
    )(a10p, a11p, hsp, d16_0, d16_1, b1t, BD)

    hs2 = hs2p.reshape(Np, O8)
    a20, a21 = _make_scatter_kernel(O8, Ep, Np, "gcn_scatter2")(
        src_p, dst_p, hs2, zerosO)

    a20p = a20.reshape(Np * O8).reshape(NP8, 128)
    a21p = a21.reshape(Np * O8).reshape(NP8, 128)
    hs2pp = hs2.reshape(Np * O8).reshape(NP8, 128)
    d8_0 = jnp.repeat(deg0, O8).reshape(NP8, 128)
    d8_1 = jnp.repeat(deg1, O8).reshape(NP8, 128)
    b2p = jnp.pad(b2, (0, O8 - O))
    b2t = jnp.tile(b2p, 128 // O8)
    Mones = jnp.kron(jnp.eye(128 // O8, dtype=jnp.float32),
                     jnp.ones((O8, O8), jnp.float32))
    BF = 896
    nbf = NP8 // BF
    outp = pl.pallas_call(
        functools.partial(_fin_body, nvalid=O, gw=O8),
        grid=(nbf,),
        in_specs=[
            pl.BlockSpec((BF, 128), lambda i: (i, 0)),
            pl.BlockSpec((BF, 128), lambda i: (i, 0)),
            pl.BlockSpec((BF, 128), lambda i: (i, 0)),
            pl.BlockSpec((BF, 128), lambda i: (i, 0)),
            pl.BlockSpec((BF, 128), lambda i: (i, 0)),
            pl.BlockSpec((128,), lambda i: (0,)),
            pl.BlockSpec((128, 128), lambda i: (0, 0)),
        ],
        out_specs=pl.BlockSpec((BF, 128), lambda i: (i, 0)),
        out_shape=jax.ShapeDtypeStruct((NP8, 128), jnp.float32),
    )(a20p, a21p, hs2pp, d8_0, d8_1, b2t, Mones)
    return outp.reshape(Np, O8)[:N, :O]


# speedup vs baseline: 1.0187x; 1.0187x over previous
import functools

import jax
import jax.numpy as jnp
from jax import lax
from jax.experimental import pallas as pl
from jax.experimental.pallas import tpu as pltpu
from jax.experimental.pallas import tpu_sc as plsc

NC = 2
NS = 16
NW = NC * NS
CH = 128
KJ = 8
BN = 1024


def _sc_mesh():
    return plsc.VectorSubcoreMesh(core_axis_name="c", subcore_axis_name="s",
                                  num_cores=NC, num_subcores=NS)


def _make_deg_kernel(Ep, Np):
    CPT = Ep // CH // NW
    SUP = CPT // KJ
    SUPP = SUP // 2
    SP = Np // NS

    @functools.partial(
        pl.kernel,
        out_type=[jax.ShapeDtypeStruct((Np,), jnp.float32)] * NC,
        mesh=_sc_mesh(),
        scratch_types=[
            pltpu.VMEM((KJ, CH), jnp.int32),
            pltpu.VMEM((KJ, CH), jnp.int32),
            pltpu.VMEM((CH,), jnp.float32),
            pltpu.VMEM((SP,), jnp.float32),
            pltpu.VMEM_SHARED((Np,), jnp.float32),
            pltpu.SemaphoreType.DMA,
            pltpu.SemaphoreType.DMA,
        ],
        compiler_params=pltpu.CompilerParams(use_tc_tiling_on_sc=False),
        name="gcn_deg",
    )
    def degk(dst_hbm, out0_hbm, out1_hbm, idxe, idxd, ones_v, stage, acc,
             seme, semd):
        c = lax.axis_index("c")
        s = lax.axis_index("s")
        wid = c * NS + s
        base = wid * CPT
        for i in range(CH // 16):
            ones_v[pl.ds(i * 16, 16)] = jnp.ones((16,), jnp.float32)

        def zbody(i, carry):
            stage[pl.ds(i * 16, 16)] = jnp.zeros((16,), jnp.float32)
            return carry

        lax.fori_loop(0, SP // 16, zbody, 0)
        pltpu.sync_copy(stage, acc.at[pl.ds(s * SP, SP)])
        plsc.subcore_barrier()

        pltpu.sync_copy(dst_hbm.at[pl.ds(base, KJ)], idxe)

        def drain(sem):
            pltpu.make_async_copy(out0_hbm.at[pl.ds(0, KJ * CH)],
                                  stage.at[pl.ds(0, KJ * CH)], sem).wait()

        def body(p, carry):
            for j in range(KJ):
                pltpu.async_copy(ones_v, acc.at[idxe.at[j]], seme, add=True)
            pltpu.sync_copy(dst_hbm.at[pl.ds(base + (2 * p + 1) * KJ, KJ)],
                            idxd)
            drain(seme)
            for j in range(KJ):
                pltpu.async_copy(ones_v, acc.at[idxd.at[j]], semd, add=True)

            @pl.when(p < SUPP - 1)
            def _():
                pltpu.sync_copy(
                    dst_hbm.at[pl.ds(base + (2 * p + 2) * KJ, KJ)], idxe)

            drain(semd)
            return carry

        lax.fori_loop(0, SUPP, body, 0)
        plsc.subcore_barrier()
        pltpu.sync_copy(acc.at[pl.ds(s * SP, SP)], stage)

        @pl.when(c == 0)
        def _():
            pltpu.sync_copy(stage, out0_hbm.at[pl.ds(s * SP, SP)])

        @pl.when(c == 1)
        def _():
            pltpu.sync_copy(stage, out1_hbm.at[pl.ds(s * SP, SP)])

    return degk


def _make_scatter_kernel(D, Ep, Np, name):
    CPT = Ep // CH // NW
    SUP = CPT // KJ
    SUPP = SUP // 2
    SP = Np // NS
    SPH = SP // 16
    RB = KJ * CH

    @functools.partial(
        pl.kernel,
        out_type=[jax.ShapeDtypeStruct((Np, D), jnp.float32)] * NC,
        mesh=_sc_mesh(),
        scratch_types=[
            pltpu.VMEM((KJ, CH), jnp.int32),
            pltpu.VMEM((KJ, CH), jnp.int32),
            pltpu.VMEM((KJ, CH), jnp.int32),
            pltpu.VMEM((KJ, CH), jnp.int32),
            pltpu.VMEM((RB, D), jnp.float32),
            pltpu.VMEM((SPH, D), jnp.float32),
            pltpu.VMEM_SHARED((Np, D), jnp.float32),
            pltpu.SemaphoreType.DMA,
            pltpu.SemaphoreType.DMA,
        ],
        compiler_params=pltpu.CompilerParams(use_tc_tiling_on_sc=False),
        name=name,
    )
    def scat(src_hbm, dst_hbm, tbl_hbm, zeros_hbm, out0_hbm, out1_hbm,
             sie, die, sid, did, rows, stage, acc, gsem, ssem):
        c = lax.axis_index("c")
        s = lax.axis_index("s")
        wid = c * NS + s
        base = wid * CPT

        def ibody(p, carry):
            sl = pl.ds(s * SP + p * SPH, SPH)
            pltpu.sync_copy(zeros_hbm.at[sl], stage)
            pltpu.sync_copy(stage, acc.at[sl])
            return carry

        lax.fori_loop(0, 16, ibody, 0)
        plsc.subcore_barrier()

        pltpu.sync_copy(src_hbm.at[pl.ds(base, KJ)], sie)
        pltpu.sync_copy(dst_hbm.at[pl.ds(base, KJ)], die)

        def gather_fire(si):
            return [pltpu.async_copy(tbl_hbm.at[si.at[j]],
                                     rows.at[pl.ds(j * CH, CH)], gsem)
                    for j in range(KJ)]

        def scatter_fire(di):
            return [pltpu.async_copy(rows.at[pl.ds(j * CH, CH)],
                                     acc.at[di.at[j]], ssem, add=True)
                    for j in range(KJ)]

        def drain(sem):
            pltpu.make_async_copy(tbl_hbm.at[pl.ds(0, RB)], rows, sem).wait()

        def phase(si, di, nsi, ndi, nxt_row, do_load):
            gather_fire(si)

            def loads():
                pltpu.sync_copy(src_hbm.at[nxt_row], nsi)
                pltpu.sync_copy(dst_hbm.at[nxt_row], ndi)

            if do_load is True:
                loads()
            else:
                pl.when(do_load)(loads)

            drain(gsem)
            scatter_fire(di)
            drain(ssem)

        def body(p, carry):
            phase(sie, die, sid, did,
                  pl.ds(base + (2 * p + 1) * KJ, KJ), True)
            phase(sid, did, sie, die,
                  pl.ds(base + (2 * p + 2) * KJ, KJ), p < SUPP - 1)
            return carry

        lax.fori_loop(0, SUPP, body, 0)
        plsc.subcore_barrier()

        def obody(p, carry):
            sl = pl.ds(s * SP + p * SPH, SPH)
            pltpu.sync_copy(acc.at[sl], stage)

            @pl.when(c == 0)
            def _():
                pltpu.sync_copy(stage, out0_hbm.at[sl])

            @pl.when(c == 1)
            def _():
                pltpu.sync_copy(stage, out1_hbm.at[sl])
            return carry

        lax.fori_loop(0, 16, obody, 0)

    return scat


def _mm_body(xt_ref, w_ref, d0_ref, d1_ref, hs_ref):
    h = lax.dot_general(xt_ref[...], w_ref[...], (((0,), (0,)), ((), ())),
                        preferred_element_type=jnp.float32)
    deg = d0_ref[...] + d1_ref[...] + 1.0
    hs_ref[...] = h * lax.rsqrt(deg).reshape(-1, 1)


def _mid_body(a0_ref, a1_ref, hs_ref, d0_ref, d1_ref, b1t_ref, bd_ref,
              hs2_ref, *, npack):
    dinv16 = lax.rsqrt(d0_ref[...] + d1_ref[...] + 1.0)
    agg = a0_ref[...] + a1_ref[...] + hs_ref[...]
    z = jnp.maximum(agg * dinv16 + b1t_ref[...][None, :], 0.0)
    h2 = jnp.dot(z, bd_ref[...], preferred_element_type=jnp.float32)
    H16 = dinv16.shape[1] // npack
    O8 = h2.shape[1] // npack
    dinv8 = jnp.concatenate(
        [dinv16[:, a * H16:a * H16 + O8] for a in range(npack)], axis=1)
    hs2_ref[...] = h2 * dinv8


def _fin_body(a0_ref, a1_ref, hs2_ref, d0_ref, d1_ref, b2t_ref, gsum_ref,
              out_ref, *, nvalid, gw):
    dinv8 = lax.rsqrt(d0_ref[...] + d1_ref[...] + 1.0)
    agg = a0_ref[...] + a1_ref[...] + hs2_ref[...]
    logits = agg * dinv8 + b2t_ref[...][None, :]
    col = lax.broadcasted_iota(jnp.int32, logits.shape, 1)
    valid = (col % gw) < nvalid
    m = jnp.max(jnp.where(valid, logits, jnp.float32(-1e30)), axis=1,
                keepdims=True)
    e = jnp.where(valid, jnp.exp(logits - m), 0.0)
    gs = jnp.dot(e, gsum_ref[...], preferred_element_type=jnp.float32)
    out_ref[...] = logits - (jnp.log(gs) + m)


def kernel(x, edge_index, W1, b1, W2, b2):
    N, F = x.shape
    H = W1.shape[1]
    O = W2.shape[1]
    E = edge_index.shape[1]
    O8 = 8

    grain = NW * CH * KJ * 2
    Ep = ((E + grain - 1) // grain) * grain
    pad = Ep - E
    Np = ((N + NW + 1023) // 1024) * 1024
    nb = Np // BN

    src = edge_index[0]
    dst = edge_index[1]
    src_p = jnp.concatenate(
        [src, jnp.zeros((pad,), jnp.int32)]).reshape(Ep // CH, CH)
    dst_p = jnp.concatenate(
        [dst, N + (jnp.arange(pad, dtype=jnp.int32) % NW)]).reshape(Ep // CH, CH)
    zerosH = jnp.zeros((Np, H), jnp.float32)
    zerosO = jnp.zeros((Np, O8), jnp.float32)

    deg0, deg1 = _make_deg_kernel(Ep, Np)(dst_p)

    xt = jnp.swapaxes(x, 0, 1)
    hs = pl.pallas_call(
        _mm_body,
        grid=(nb,),
        in_specs=[
            pl.BlockSpec((F, BN), lambda i: (0, i)),
            pl.BlockSpec((F, H), lambda i: (0, 0)),
            pl.BlockSpec((BN,), lambda i: (i,)),
            pl.BlockSpec((BN,), lambda i: (i,)),
        ],
        out_specs=pl.BlockSpec((BN, H), lambda i: (i, 0)),
        out_shape=jax.ShapeDtypeStruct((Np, H), jnp.float32),
    )(xt, W1, deg0, deg1)

    a10, a11 = _make_scatter_kernel(H, Ep, Np, "gcn_scatter1")(
        src_p, dst_p, hs, zerosH)

    NP16 = Np * H // 128
    NP8 = Np * O8 // 128
    B16 = BN * H // 128
    B8 = BN * O8 // 128
    hsp = hs.reshape(Np * H).reshape(NP16, 128)
    a10p = a10.reshape(Np * H).reshape(NP16, 128)
    a11p = a11.reshape(Np * H).reshape(NP16, 128)
    d16_0 = jnp.repeat(deg0, H).reshape(NP16, 128)
    d16_1 = jnp.repeat(deg1, H).reshape(NP16, 128)

    W2p = jnp.pad(W2, ((0, 0), (0, O8 - O)))
    BD = jnp.kron(jnp.eye(128 // H, dtype=jnp.float32), W2p)
    b1t = jnp.tile(b1, 128 // H)
    BM = 896
    nbm = NP16 // BM
    hs2p = pl.pallas_call(
        functools.partial(_mid_body, npack=128 // H),
        grid=(nbm,),
        in_specs=[
            pl.BlockSpec((BM, 128), lambda i: (i, 0)),
            pl.BlockSpec((BM, 128), lambda i: (i, 0)),
            pl.BlockSpec((BM, 128), lambda i: (i, 0)),
            pl.BlockSpec((BM, 128), lambda i: (i, 0)),
            pl.BlockSpec((BM, 128), lambda i: (i, 0)),
            pl.BlockSpec((128,), lambda i: (0,)),
            pl.BlockSpec((128, 128 // H * O8), lambda i: (0, 0)),
        ],
        out_specs=pl.BlockSpec((BM, 128 // H * O8), lambda i: (i, 0)),
        out_shape=jax.ShapeDtypeStruct((NP16, 128 // H * O8), jnp.float32),
    )(a10p, a11p, hsp, d16_0, d16_1, b1t, BD)

    hs2 = hs2p.reshape(Np, O8)
    a20, a21 = _make_scatter_kernel(O8, Ep, Np, "gcn_scatter2")(
        src_p, dst_p, hs2, zerosO)

    a20p = a20.reshape(Np * O8).reshape(NP8, 128)
    a21p = a21.reshape(Np * O8).reshape(NP8, 128)
    hs2pp = hs2.reshape(Np * O8).reshape(NP8, 128)
    d8_0 = jnp.repeat(deg0, O8).reshape(NP8, 128)
    d8_1 = jnp.repeat(deg1, O8).reshape(NP8, 128)
    b2p = jnp.pad(b2, (0, O8 - O))
    b2t = jnp.tile(b2p, 128 // O8)
    Mones = jnp.kron(jnp.eye(128 // O8, dtype=jnp.float32),
                     jnp.ones((O8, O8), jnp.float32))
    BF = 896
    nbf = NP8 // BF
    outp = pl.pallas_call(
        functools.partial(_fin_body, nvalid=O, gw=O8),
        grid=(nbf,),
        in_specs=[
            pl.BlockSpec((BF, 128), lambda i: (i, 0)),
            pl.BlockSpec((BF, 128), lambda i: (i, 0)),
            pl.BlockSpec((BF, 128), lambda i: (i, 0)),
            pl.BlockSpec((BF, 128), lambda i: (i, 0)),
            pl.BlockSpec((BF, 128), lambda i: (i, 0)),
            pl.BlockSpec((128,), lambda i: (0,)),
            pl.BlockSpec((128, 128), lambda i: (0, 0)),
        ],
        out_specs=pl.BlockSpec((BF, 128), lambda i: (i, 0)),
        out_shape=jax.ShapeDtypeStruct((NP8, 128), jnp.float32),
    )(a20p, a21p, hs2pp, d8_0, d8_1, b2t, Mones)
    return outp.reshape(Np, O8)[:N, :O]
